# baseline (speedup 1.0000x reference)
import functools
import jax
import jax.numpy as jnp
from jax import lax
from jax.experimental import pallas as pl
from jax.experimental.pallas import tpu as pltpu, tpu_sc as plsc

B = 4
S = 2048
D2 = 1024
N = B * S
NC, NS, L = 2, 16, 16
NW = NC * NS
ROWS_PER_W = N // NW
R = 16
NCHUNK = ROWS_PER_W // R


def _sc_body(x_hbm, par_hbm, pe_hbm, out_hbm, idx_v, xbuf, posbuf, parbuf, gsem):
    wid = lax.axis_index("s") * NC + lax.axis_index("c")
    base = wid * ROWS_PER_W
    s0 = lax.rem(base, S)

    def chunk(c, carry):
        row0 = base + c * R
        pltpu.sync_copy(par_hbm.at[pl.ds(row0, R)], idx_v)
        g = pltpu.async_copy(pe_hbm.at[idx_v], parbuf, gsem)
        pltpu.sync_copy(x_hbm.at[pl.ds(row0, R)], xbuf)
        pltpu.sync_copy(pe_hbm.at[pl.ds(s0 + c * R, R)], posbuf)
        g.wait()

        def row(i, carry2):
            for j in range(D2 // L):
                sl = pl.ds(j * L, L)
                xbuf[i, 0, sl] += posbuf[i, sl]
                xbuf[i, 1, sl] += parbuf[i, sl]
            return carry2

        lax.fori_loop(0, R, row, 0, unroll=False)
        pltpu.sync_copy(xbuf, out_hbm.at[pl.ds(row0, R)])
        return carry

    lax.fori_loop(0, NCHUNK, chunk, 0, unroll=False)


@jax.jit
def _sc_call(x2, par, pe0):
    mesh = plsc.VectorSubcoreMesh(core_axis_name="c", subcore_axis_name="s")
    f = pl.kernel(
        _sc_body,
        out_type=jax.ShapeDtypeStruct((N, 2, D2), jnp.float32),
        mesh=mesh,
        scratch_types=[
            pltpu.VMEM((R,), jnp.int32),
            pltpu.VMEM((R, 2, D2), jnp.float32),
            pltpu.VMEM((R, D2), jnp.float32),
            pltpu.VMEM((R, D2), jnp.float32),
            pltpu.SemaphoreType.DMA,
        ],
    )
    return f(x2, par, pe0)


def kernel(x, parents, pe):
    x2 = x.reshape(N, 2, D2)
    par = parents.reshape(N).astype(jnp.int32)
    pe0 = pe[0]
    out = _sc_call(x2, par, pe0)
    return out.reshape(B, S, 2 * D2)


# speedup vs baseline: 1.2157x; 1.2157x over previous
import jax
import jax.numpy as jnp
from jax import lax
from jax.experimental import pallas as pl
from jax.experimental.pallas import tpu as pltpu, tpu_sc as plsc

B = 4
S = 2048
D2 = 1024
N = B * S
NC, NS, L = 2, 16, 16
NW = NC * NS
ROWS_PER_W = N // NW
R = 8
NCHUNK = ROWS_PER_W // R
NGRP = NCHUNK // 2


def _sc_body(x_hbm, par_hbm, pe_hbm, out_hbm,
             pidx, xbuf0, xbuf1, pbuf0, pbuf1, gbuf0, gbuf1,
             sl0, sl1, so0, so1):
    wid = lax.axis_index("s") * NC + lax.axis_index("c")
    base = wid * ROWS_PER_W
    s0 = lax.rem(base, S)

    xbuf = (xbuf0, xbuf1)
    pbuf = (pbuf0, pbuf1)
    gbuf = (gbuf0, gbuf1)
    sl = (sl0, sl1)
    so = (so0, so1)

    pltpu.sync_copy(par_hbm.at[pl.ds(base, ROWS_PER_W)], pidx)

    def loads(c, bsel):
        rows = pl.ds(base + c * R, R)
        pltpu.make_async_copy(x_hbm.at[rows], xbuf[bsel], sl[bsel]).start()
        pltpu.make_async_copy(pe_hbm.at[pl.ds(s0 + c * R, R)], pbuf[bsel], sl[bsel]).start()
        pltpu.make_async_copy(pe_hbm.at[pidx.at[pl.ds(c * R, R)]], gbuf[bsel], sl[bsel]).start()

    def process(c, bsel):
        rows = pl.ds(base + c * R, R)
        pltpu.make_async_copy(x_hbm.at[rows], xbuf[bsel], sl[bsel]).wait()
        pltpu.make_async_copy(pe_hbm.at[pl.ds(s0 + c * R, R)], pbuf[bsel], sl[bsel]).wait()
        pltpu.make_async_copy(pe_hbm.at[pidx.at[pl.ds(c * R, R)]], gbuf[bsel], sl[bsel]).wait()

        def row(i, carry):
            for j in range(D2 // L):
                cols = pl.ds(j * L, L)
                plsc.addupdate(xbuf[bsel].at[i, 0, cols], pbuf[bsel][i, cols])
                plsc.addupdate(xbuf[bsel].at[i, 1, cols], gbuf[bsel][i, cols])
            return carry

        lax.fori_loop(0, R, row, 0, unroll=False)
        pltpu.make_async_copy(xbuf[bsel], out_hbm.at[rows], so[bsel]).start()

    def store_wait(c, bsel):
        rows = pl.ds(base + c * R, R)
        pltpu.make_async_copy(xbuf[bsel], out_hbm.at[rows], so[bsel]).wait()

    loads(0, 0)
    loads(1, 1)

    def group(g, carry):
        for bsel in (0, 1):
            c = 2 * g + bsel
            process(c, bsel)
        for bsel in (0, 1):
            c = 2 * g + bsel
            store_wait(c, bsel)
            loads(c + 2, bsel)
        return carry

    lax.fori_loop(0, NGRP - 1, group, 0, unroll=False)

    for bsel in (0, 1):
        process(NCHUNK - 2 + bsel, bsel)
    for bsel in (0, 1):
        store_wait(NCHUNK - 2 + bsel, bsel)


@jax.jit
def _sc_call(x2, par, pe0):
    mesh = plsc.VectorSubcoreMesh(core_axis_name="c", subcore_axis_name="s")
    f = pl.kernel(
        _sc_body,
        out_type=jax.ShapeDtypeStruct((N, 2, D2), jnp.float32),
        mesh=mesh,
        scratch_types=[
            pltpu.VMEM((ROWS_PER_W,), jnp.int32),
            pltpu.VMEM((R, 2, D2), jnp.float32),
            pltpu.VMEM((R, 2, D2), jnp.float32),
            pltpu.VMEM((R, D2), jnp.float32),
            pltpu.VMEM((R, D2), jnp.float32),
            pltpu.VMEM((R, D2), jnp.float32),
            pltpu.VMEM((R, D2), jnp.float32),
            pltpu.SemaphoreType.DMA,
            pltpu.SemaphoreType.DMA,
            pltpu.SemaphoreType.DMA,
            pltpu.SemaphoreType.DMA,
        ],
    )
    return f(x2, par, pe0)


def kernel(x, parents, pe):
    x2 = x.reshape(N, 2, D2)
    par = parents.reshape(N).astype(jnp.int32)
    pe0 = pe[0]
    out = _sc_call(x2, par, pe0)
    return out.reshape(B, S, 2 * D2)


# speedup vs baseline: 3.4268x; 2.8188x over previous
import jax
import jax.numpy as jnp
from jax import lax
from jax.experimental import pallas as pl
from jax.experimental.pallas import tpu as pltpu, tpu_sc as plsc

B = 4
S = 2048
D2 = 1024
D = 2 * D2
N = B * S
NC, NS, L = 2, 16, 16
NW = NC * NS
WPB = NW // B
ROWS_PER_W = S // WPB
R = 8
NCHUNK = ROWS_PER_W // R
NGRP = NCHUNK // 2


def _sc_body(x_hbm, par_hbm, pe_hbm, out_hbm,
             pidx, xbuf0, xbuf1, pbuf0, pbuf1, gbuf0, gbuf1,
             sl0, sl1, so0, so1):
    wid = lax.axis_index("s") * NC + lax.axis_index("c")
    bidx = wid // WPB
    s0 = (wid % WPB) * ROWS_PER_W

    xbuf = (xbuf0, xbuf1)
    pbuf = (pbuf0, pbuf1)
    gbuf = (gbuf0, gbuf1)
    sl = (sl0, sl1)
    so = (so0, so1)

    pltpu.sync_copy(par_hbm.at[bidx, pl.ds(s0, ROWS_PER_W)], pidx)

    def loads(c, bsel):
        rows = pl.ds(s0 + c * R, R)
        pltpu.make_async_copy(x_hbm.at[bidx, rows], xbuf[bsel], sl[bsel]).start()
        pltpu.make_async_copy(pe_hbm.at[pl.ds(s0 + c * R, R)], pbuf[bsel], sl[bsel]).start()
        pltpu.make_async_copy(pe_hbm.at[pidx.at[pl.ds(c * R, R)]], gbuf[bsel], sl[bsel]).start()

    def process(c, bsel):
        rows = pl.ds(s0 + c * R, R)
        pltpu.make_async_copy(x_hbm.at[bidx, rows], xbuf[bsel], sl[bsel]).wait()
        pltpu.make_async_copy(pe_hbm.at[pl.ds(s0 + c * R, R)], pbuf[bsel], sl[bsel]).wait()
        pltpu.make_async_copy(pe_hbm.at[pidx.at[pl.ds(c * R, R)]], gbuf[bsel], sl[bsel]).wait()

        def row(i, carry):
            for j in range(D2 // L):
                cols = pl.ds(j * L, L)
                hi = pl.ds(D2 + j * L, L)
                plsc.addupdate(xbuf[bsel].at[i, cols], pbuf[bsel][i, cols])
                plsc.addupdate(xbuf[bsel].at[i, hi], gbuf[bsel][i, cols])
            return carry

        lax.fori_loop(0, R, row, 0, unroll=False)
        pltpu.make_async_copy(xbuf[bsel], out_hbm.at[bidx, rows], so[bsel]).start()

    def store_wait(c, bsel):
        rows = pl.ds(s0 + c * R, R)
        pltpu.make_async_copy(xbuf[bsel], out_hbm.at[bidx, rows], so[bsel]).wait()

    loads(0, 0)
    loads(1, 1)

    def group(g, carry):
        for bsel in (0, 1):
            process(2 * g + bsel, bsel)
        for bsel in (0, 1):
            store_wait(2 * g + bsel, bsel)
            loads(2 * g + bsel + 2, bsel)
        return carry

    lax.fori_loop(0, NGRP - 1, group, 0, unroll=False)

    for bsel in (0, 1):
        process(NCHUNK - 2 + bsel, bsel)
    for bsel in (0, 1):
        store_wait(NCHUNK - 2 + bsel, bsel)


@jax.jit
def _sc_call(x, par, pe0):
    mesh = plsc.VectorSubcoreMesh(core_axis_name="c", subcore_axis_name="s")
    f = pl.kernel(
        _sc_body,
        out_type=jax.ShapeDtypeStruct((B, S, D), jnp.float32),
        mesh=mesh,
        scratch_types=[
            pltpu.VMEM((ROWS_PER_W,), jnp.int32),
            pltpu.VMEM((R, D), jnp.float32),
            pltpu.VMEM((R, D), jnp.float32),
            pltpu.VMEM((R, D2), jnp.float32),
            pltpu.VMEM((R, D2), jnp.float32),
            pltpu.VMEM((R, D2), jnp.float32),
            pltpu.VMEM((R, D2), jnp.float32),
            pltpu.SemaphoreType.DMA,
            pltpu.SemaphoreType.DMA,
            pltpu.SemaphoreType.DMA,
            pltpu.SemaphoreType.DMA,
        ],
    )
    return f(x, par, pe0)


def kernel(x, parents, pe):
    par = parents.astype(jnp.int32)
    pe0 = pe[0]
    return _sc_call(x, par, pe0)
